# baseline (speedup 1.0000x reference)
import functools

import jax
import jax.numpy as jnp
from jax import lax
from jax.experimental import pallas as pl
from jax.experimental.pallas import tpu as pltpu
from jax.experimental.pallas import tpu_sc as plsc

_NUM_UNIQ = [1000000, 1000000]
_D = 32
_F = 2

_NC = 2
_NS = 16
_NW = _NC * _NS
_LANES = 16

_SUB = 128


def _sc_convert(table):
    v_rows = table.shape[0]
    slab_cols = 640
    n_slabs = v_rows // slab_cols
    per_w = -(-n_slabs // _NW)
    slab_out = slab_cols * _D
    tt = table.T

    mesh = plsc.VectorSubcoreMesh(core_axis_name="c", subcore_axis_name="s")

    @functools.partial(
        pl.kernel,
        mesh=mesh,
        out_type=jax.ShapeDtypeStruct((v_rows * _D,), jnp.float32),
        scratch_types=[
            pltpu.VMEM((_D, slab_cols + 3), jnp.float32),
            pltpu.VMEM((slab_out,), jnp.float32),
            pltpu.VMEM((slab_out,), jnp.float32),
            pltpu.SemaphoreType.DMA,
        ],
        compiler_params=pltpu.CompilerParams(use_tc_tiling_on_sc=True,
                                             needs_layout_passes=False),
    )
    def conv_kernel(tt_hbm, out_hbm, in_v, out_a, out_b, sem):
        wid = lax.axis_index("s") * _NC + lax.axis_index("c")
        lanes = lax.iota(jnp.int32, 16)
        out_bufs = (out_a, out_b)

        def do_slab(slab, out_v):
            col0 = pl.multiple_of(slab * slab_cols, slab_cols)
            pltpu.sync_copy(tt_hbm.at[:, pl.ds(col0, slab_cols)],
                            in_v.at[:, pl.ds(0, slab_cols)])

            def row_body(rb, carry2):
                for u in range(8):
                    ri = rb * 8 + u
                    ri_vec = jnp.full((16,), ri, jnp.int32)
                    for h in range(2):
                        vals = plsc.load_gather(
                            in_v, [h * _LANES + lanes, ri_vec])
                        out_v[pl.ds(
                            pl.multiple_of(ri * _D + h * _LANES, _LANES),
                            _LANES)] = vals
                return carry2

            lax.fori_loop(0, slab_cols // 8, row_body, 0)
            return pltpu.async_copy(
                out_v,
                out_hbm.at[pl.ds(pl.multiple_of(slab * slab_out, 1024),
                                 slab_out)],
                sem)

        def wait_out(i, buf):
            prev_slab = i * _NW + wid
            pltpu.make_async_copy(
                buf,
                out_hbm.at[pl.ds(
                    pl.multiple_of(prev_slab * slab_out, 1024), slab_out)],
                sem).wait()

        def it_body(k, carry):
            for u in range(2):
                i = k * 2 + u
                buf = out_bufs[u]
                slab = i * _NW + wid

                @pl.when(slab < n_slabs)
                def _(i=i, buf=buf, slab=slab):
                    @pl.when(i >= 2)
                    def _():
                        wait_out(i - 2, buf)
                    do_slab(slab, buf)
            return carry

        lax.fori_loop(0, per_w // 2, it_body, 0)
        for i in range(max(per_w - 3, 0), per_w):
            slab = i * _NW + wid
            nxt = (i + 2) * _NW + wid

            @pl.when((slab < n_slabs) & (nxt >= n_slabs))
            def _(i=i):
                wait_out(i, out_bufs[i % 2])

    return conv_kernel(tt).reshape(v_rows, _D)


def _sc_gather(x_flat, table_l, rows_total, chunk, field1_off):
    per_w = rows_total // _NW
    n_chunks = per_w // chunk
    n_sub = chunk // _SUB
    n_vec = chunk // _LANES

    mesh = plsc.VectorSubcoreMesh(core_axis_name="c", subcore_axis_name="s")

    @functools.partial(
        pl.kernel,
        mesh=mesh,
        out_type=jax.ShapeDtypeStruct((rows_total, _D), jnp.float32),
        scratch_types=[
            pltpu.VMEM((chunk,), jnp.int32),
            pltpu.VMEM((chunk, _D), jnp.float32),
            pltpu.SemaphoreType.DMA,
        ],
        compiler_params=pltpu.CompilerParams(use_tc_tiling_on_sc=False),
    )
    def gather_kernel(table_hbm, idx_hbm, out_hbm, idx_v, rows_v, sem):
        wid = lax.axis_index("s") * _NC + lax.axis_index("c")
        base = wid * per_w
        pat = (lax.iota(jnp.int32, 16) & 1) * field1_off

        def chunk_body(i, carry):
            off = pl.multiple_of(base + i * chunk, _SUB)
            pltpu.sync_copy(idx_hbm.at[pl.ds(off, chunk)], idx_v)
            for j in range(n_vec):
                sl = pl.ds(j * _LANES, _LANES)
                idx_v[sl] = idx_v[sl] + pat
            handles = []
            for j in range(n_sub):
                handles.append(
                    pltpu.async_copy(
                        table_hbm.at[idx_v.at[pl.ds(j * _SUB, _SUB)]],
                        rows_v.at[pl.ds(j * _SUB, _SUB)],
                        sem,
                    )
                )
            for h in handles:
                h.wait()
            pltpu.sync_copy(rows_v, out_hbm.at[pl.ds(off, chunk)])
            return carry

        lax.fori_loop(0, n_chunks, chunk_body, 0)

    return gather_kernel(table_l, x_flat)


def _tc_project(emb4, w4, b4, tile_n):
    n4, fd4 = emb4.shape
    d4 = w4.shape[1]

    def mm_kernel(emb_ref, w_ref, b_ref, out_ref):
        out_ref[...] = (
            jnp.dot(emb_ref[...], w_ref[...],
                    preferred_element_type=jnp.float32)
            + b_ref[...]
        )

    return pl.pallas_call(
        mm_kernel,
        grid=(n4 // tile_n,),
        in_specs=[
            pl.BlockSpec((tile_n, fd4), lambda i: (i, 0)),
            pl.BlockSpec((fd4, d4), lambda i: (0, 0)),
            pl.BlockSpec((1, d4), lambda i: (0, 0)),
        ],
        out_specs=pl.BlockSpec((tile_n, d4), lambda i: (i, 0)),
        out_shape=jax.ShapeDtypeStruct((n4, d4), jnp.float32),
    )(emb4, w4, b4)


def kernel(x, table, W, b):
    B, S, G, F = x.shape
    n_rows = B * S * G
    rows_total = n_rows * F

    x_flat = x.reshape(rows_total)
    table_l = _sc_convert(table)
    gathered = _sc_gather(x_flat, table_l, rows_total, chunk=1280,
                          field1_off=_NUM_UNIQ[0])
    emb4 = gathered.reshape(n_rows // 4, 4 * F * _D)
    wt = W.T
    z = jnp.zeros_like(wt)
    w4 = jnp.block([
        [wt, z, z, z],
        [z, wt, z, z],
        [z, z, wt, z],
        [z, z, z, wt],
    ])
    b4 = jnp.tile(b, 4).reshape(1, 4 * _D)
    out4 = _tc_project(emb4, w4, b4, tile_n=1024)
    return out4.reshape(B, S, G, _D)


# speedup vs baseline: 1.1685x; 1.1685x over previous
import functools

import jax
import jax.numpy as jnp
from jax import lax
from jax.experimental import pallas as pl
from jax.experimental.pallas import tpu as pltpu
from jax.experimental.pallas import tpu_sc as plsc

_NUM_UNIQ = [1000000, 1000000]
_D = 32
_F = 2

_NC = 2
_NS = 16
_NW = _NC * _NS
_LANES = 16

_SUB = 128


def _sc_convert(table):
    v_rows = table.shape[0]
    slab_cols = 640
    n_slabs = v_rows // slab_cols
    per_w = -(-n_slabs // _NW)
    slab_out = slab_cols * _D
    tt = table.T

    mesh = plsc.VectorSubcoreMesh(core_axis_name="c", subcore_axis_name="s")

    @functools.partial(
        pl.kernel,
        mesh=mesh,
        out_type=jax.ShapeDtypeStruct((v_rows * _D,), jnp.float32),
        scratch_types=[
            pltpu.VMEM((_D, slab_cols), jnp.float32),
            pltpu.VMEM((slab_out,), jnp.float32),
            pltpu.VMEM((slab_out,), jnp.float32),
            pltpu.SemaphoreType.DMA,
        ],
        compiler_params=pltpu.CompilerParams(use_tc_tiling_on_sc=True,
                                             needs_layout_passes=False),
    )
    def conv_kernel(tt_hbm, out_hbm, in_v, out_a, out_b, sem):
        wid = lax.axis_index("s") * _NC + lax.axis_index("c")
        lanes = lax.iota(jnp.int32, 16)
        out_bufs = (out_a, out_b)

        def do_slab(slab, out_v):
            col0 = pl.multiple_of(slab * slab_cols, slab_cols)
            pltpu.sync_copy(tt_hbm.at[:, pl.ds(col0, slab_cols)], in_v)

            def grp_body(j, carry2):
                riv32 = (j * _LANES + lanes) * _D
                for c in range(_D):
                    vals = in_v[c, pl.ds(
                        pl.multiple_of(j * _LANES, _LANES), _LANES)]
                    plsc.store_scatter(out_v, [riv32 + c], vals)
                return carry2

            lax.fori_loop(0, slab_cols // _LANES, grp_body, 0)
            return pltpu.async_copy(
                out_v,
                out_hbm.at[pl.ds(pl.multiple_of(slab * slab_out, 1024),
                                 slab_out)],
                sem)

        def wait_out(i, buf):
            prev_slab = i * _NW + wid
            pltpu.make_async_copy(
                buf,
                out_hbm.at[pl.ds(
                    pl.multiple_of(prev_slab * slab_out, 1024), slab_out)],
                sem).wait()

        def it_body(k, carry):
            for u in range(2):
                i = k * 2 + u
                buf = out_bufs[u]
                slab = i * _NW + wid

                @pl.when(slab < n_slabs)
                def _(i=i, buf=buf, slab=slab):
                    @pl.when(i >= 2)
                    def _():
                        wait_out(i - 2, buf)
                    do_slab(slab, buf)
            return carry

        lax.fori_loop(0, per_w // 2, it_body, 0)
        for i in range(max(per_w - 3, 0), per_w):
            slab = i * _NW + wid
            nxt = (i + 2) * _NW + wid

            @pl.when((slab < n_slabs) & (nxt >= n_slabs))
            def _(i=i):
                wait_out(i, out_bufs[i % 2])

    return conv_kernel(tt).reshape(v_rows, _D)


def _sc_gather(x_flat, table_l, rows_total, chunk, field1_off):
    per_w = rows_total // _NW
    n_chunks = per_w // chunk
    n_sub = chunk // _SUB
    n_vec = chunk // _LANES

    mesh = plsc.VectorSubcoreMesh(core_axis_name="c", subcore_axis_name="s")

    @functools.partial(
        pl.kernel,
        mesh=mesh,
        out_type=jax.ShapeDtypeStruct((rows_total, _D), jnp.float32),
        scratch_types=[
            pltpu.VMEM((chunk,), jnp.int32),
            pltpu.VMEM((chunk, _D), jnp.float32),
            pltpu.SemaphoreType.DMA,
        ],
        compiler_params=pltpu.CompilerParams(use_tc_tiling_on_sc=False),
    )
    def gather_kernel(table_hbm, idx_hbm, out_hbm, idx_v, rows_v, sem):
        wid = lax.axis_index("s") * _NC + lax.axis_index("c")
        base = wid * per_w
        pat = (lax.iota(jnp.int32, 16) & 1) * field1_off

        def chunk_body(i, carry):
            off = pl.multiple_of(base + i * chunk, _SUB)
            pltpu.sync_copy(idx_hbm.at[pl.ds(off, chunk)], idx_v)
            for j in range(n_vec):
                sl = pl.ds(j * _LANES, _LANES)
                idx_v[sl] = idx_v[sl] + pat
            handles = []
            for j in range(n_sub):
                handles.append(
                    pltpu.async_copy(
                        table_hbm.at[idx_v.at[pl.ds(j * _SUB, _SUB)]],
                        rows_v.at[pl.ds(j * _SUB, _SUB)],
                        sem,
                    )
                )
            for h in handles:
                h.wait()
            pltpu.sync_copy(rows_v, out_hbm.at[pl.ds(off, chunk)])
            return carry

        lax.fori_loop(0, n_chunks, chunk_body, 0)

    return gather_kernel(table_l, x_flat)


def _tc_project(emb4, w4, b4, tile_n):
    n4, fd4 = emb4.shape
    d4 = w4.shape[1]

    def mm_kernel(emb_ref, w_ref, b_ref, out_ref):
        out_ref[...] = (
            jnp.dot(emb_ref[...], w_ref[...],
                    preferred_element_type=jnp.float32)
            + b_ref[...]
        )

    return pl.pallas_call(
        mm_kernel,
        grid=(n4 // tile_n,),
        in_specs=[
            pl.BlockSpec((tile_n, fd4), lambda i: (i, 0)),
            pl.BlockSpec((fd4, d4), lambda i: (0, 0)),
            pl.BlockSpec((1, d4), lambda i: (0, 0)),
        ],
        out_specs=pl.BlockSpec((tile_n, d4), lambda i: (i, 0)),
        out_shape=jax.ShapeDtypeStruct((n4, d4), jnp.float32),
    )(emb4, w4, b4)


def kernel(x, table, W, b):
    B, S, G, F = x.shape
    n_rows = B * S * G
    rows_total = n_rows * F

    x_flat = x.reshape(rows_total)
    table_l = _sc_convert(table)
    gathered = _sc_gather(x_flat, table_l, rows_total, chunk=1280,
                          field1_off=_NUM_UNIQ[0])
    emb4 = gathered.reshape(n_rows // 4, 4 * F * _D)
    wt = W.T
    z = jnp.zeros_like(wt)
    w4 = jnp.block([
        [wt, z, z, z],
        [z, wt, z, z],
        [z, z, wt, z],
        [z, z, z, wt],
    ])
    b4 = jnp.tile(b, 4).reshape(1, 4 * _D)
    out4 = _tc_project(emb4, w4, b4, tile_n=1024)
    return out4.reshape(B, S, G, _D)


# speedup vs baseline: 1.8317x; 1.5676x over previous
import functools

import jax
import jax.numpy as jnp
from jax import lax
from jax.experimental import pallas as pl
from jax.experimental.pallas import tpu as pltpu
from jax.experimental.pallas import tpu_sc as plsc

_NUM_UNIQ = [1000000, 1000000]
_D = 32
_F = 2

_NC = 2
_NS = 16
_NW = _NC * _NS
_LANES = 16

_SUB = 128


def _sc_convert(table):
    v_rows = table.shape[0]
    slab_cols = 640
    n_slabs = v_rows // slab_cols
    per_w = -(-n_slabs // _NW)
    slab_out = slab_cols * _D
    tt = table.T

    mesh = plsc.VectorSubcoreMesh(core_axis_name="c", subcore_axis_name="s")

    @functools.partial(
        pl.kernel,
        mesh=mesh,
        out_type=jax.ShapeDtypeStruct((v_rows * _D,), jnp.float32),
        scratch_types=[
            pltpu.VMEM((_D, slab_cols), jnp.float32),
            pltpu.VMEM((slab_out,), jnp.float32),
            pltpu.VMEM((slab_out,), jnp.float32),
            pltpu.SemaphoreType.DMA,
        ],
        compiler_params=pltpu.CompilerParams(use_tc_tiling_on_sc=True,
                                             needs_layout_passes=False),
    )
    def conv_kernel(tt_hbm, out_hbm, in_v, out_a, out_b, sem):
        wid = lax.axis_index("s") * _NC + lax.axis_index("c")
        lanes = lax.iota(jnp.int32, 16)
        out_bufs = (out_a, out_b)

        def do_slab(slab, out_v):
            col0 = pl.multiple_of(slab * slab_cols, slab_cols)
            pltpu.sync_copy(tt_hbm.at[:, pl.ds(col0, slab_cols)], in_v)

            def grp_body(j, carry2):
                riv32 = (j * _LANES + lanes) * _D
                for c in range(_D):
                    vals = in_v[c, pl.ds(
                        pl.multiple_of(j * _LANES, _LANES), _LANES)]
                    plsc.store_scatter(out_v, [riv32 + c], vals)
                return carry2

            lax.fori_loop(0, slab_cols // _LANES, grp_body, 0)
            return pltpu.async_copy(
                out_v,
                out_hbm.at[pl.ds(pl.multiple_of(slab * slab_out, 1024),
                                 slab_out)],
                sem)

        def wait_out(i, buf):
            prev_slab = i * _NW + wid
            pltpu.make_async_copy(
                buf,
                out_hbm.at[pl.ds(
                    pl.multiple_of(prev_slab * slab_out, 1024), slab_out)],
                sem).wait()

        def it_body(k, carry):
            for u in range(2):
                i = k * 2 + u
                buf = out_bufs[u]
                slab = i * _NW + wid

                @pl.when(slab < n_slabs)
                def _(i=i, buf=buf, slab=slab):
                    @pl.when(i >= 2)
                    def _():
                        wait_out(i - 2, buf)
                    do_slab(slab, buf)
            return carry

        lax.fori_loop(0, per_w // 2, it_body, 0)
        for i in range(max(per_w - 3, 0), per_w):
            slab = i * _NW + wid
            nxt = (i + 2) * _NW + wid

            @pl.when((slab < n_slabs) & (nxt >= n_slabs))
            def _(i=i):
                wait_out(i, out_bufs[i % 2])

    return conv_kernel(tt).reshape(v_rows, _D)


def _tc_convert(table, block_cols=3200):
    v_rows = table.shape[0]
    tt = table.T
    n_blk = v_rows // block_cols
    rows_out = block_cols // 4
    eye = jnp.eye(_D, dtype=jnp.float32)

    def tr_kernel(tt_ref, eye_ref, out_ref):
        t = jax.lax.dot_general(
            tt_ref[...], eye_ref[...], (((0,), (0,)), ((), ())),
            preferred_element_type=jnp.float32)
        for k in range(block_cols // 128):
            for q in range(4):
                r0 = 128 * k + 32 * q
                out_ref[pl.ds(32 * k, 32), pl.ds(32 * q, 32)] = (
                    t[r0:r0 + 32, :]
                )

    out = pl.pallas_call(
        tr_kernel,
        grid=(n_blk,),
        in_specs=[
            pl.BlockSpec((_D, block_cols), lambda i: (0, i)),
            pl.BlockSpec((_D, _D), lambda i: (0, 0)),
        ],
        out_specs=pl.BlockSpec((rows_out, 4 * _D), lambda i: (i, 0)),
        out_shape=jax.ShapeDtypeStruct((v_rows // 4, 4 * _D), jnp.float32),
    )(tt, eye)
    return out.reshape(v_rows, _D)


def _sc_gather(x_flat, table_l, rows_total, chunk, field1_off):
    per_w = rows_total // _NW
    n_chunks = per_w // chunk
    n_sub = chunk // _SUB
    n_vec = chunk // _LANES

    mesh = plsc.VectorSubcoreMesh(core_axis_name="c", subcore_axis_name="s")

    @functools.partial(
        pl.kernel,
        mesh=mesh,
        out_type=jax.ShapeDtypeStruct((rows_total, _D), jnp.float32),
        scratch_types=[
            pltpu.VMEM((chunk,), jnp.int32),
            pltpu.VMEM((chunk, _D), jnp.float32),
            pltpu.SemaphoreType.DMA,
        ],
        compiler_params=pltpu.CompilerParams(use_tc_tiling_on_sc=False),
    )
    def gather_kernel(table_hbm, idx_hbm, out_hbm, idx_v, rows_v, sem):
        wid = lax.axis_index("s") * _NC + lax.axis_index("c")
        base = wid * per_w
        pat = (lax.iota(jnp.int32, 16) & 1) * field1_off

        def chunk_body(i, carry):
            off = pl.multiple_of(base + i * chunk, _SUB)
            pltpu.sync_copy(idx_hbm.at[pl.ds(off, chunk)], idx_v)
            for j in range(n_vec):
                sl = pl.ds(j * _LANES, _LANES)
                r = idx_v[sl] + pat
                idx_v[sl] = (
                    (r & ~jnp.int32(127))
                    | lax.shift_left((r & 31), 2)
                    | (lax.shift_right_logical(r, 5) & 3)
                )
            handles = []
            for j in range(n_sub):
                handles.append(
                    pltpu.async_copy(
                        table_hbm.at[idx_v.at[pl.ds(j * _SUB, _SUB)]],
                        rows_v.at[pl.ds(j * _SUB, _SUB)],
                        sem,
                    )
                )
            for h in handles:
                h.wait()
            pltpu.sync_copy(rows_v, out_hbm.at[pl.ds(off, chunk)])
            return carry

        lax.fori_loop(0, n_chunks, chunk_body, 0)

    return gather_kernel(table_l, x_flat)


def _tc_project(emb4, w4, b4, tile_n):
    n4, fd4 = emb4.shape
    d4 = w4.shape[1]

    def mm_kernel(emb_ref, w_ref, b_ref, out_ref):
        out_ref[...] = (
            jnp.dot(emb_ref[...], w_ref[...],
                    preferred_element_type=jnp.float32)
            + b_ref[...]
        )

    return pl.pallas_call(
        mm_kernel,
        grid=(n4 // tile_n,),
        in_specs=[
            pl.BlockSpec((tile_n, fd4), lambda i: (i, 0)),
            pl.BlockSpec((fd4, d4), lambda i: (0, 0)),
            pl.BlockSpec((1, d4), lambda i: (0, 0)),
        ],
        out_specs=pl.BlockSpec((tile_n, d4), lambda i: (i, 0)),
        out_shape=jax.ShapeDtypeStruct((n4, d4), jnp.float32),
    )(emb4, w4, b4)


def kernel(x, table, W, b):
    B, S, G, F = x.shape
    n_rows = B * S * G
    rows_total = n_rows * F

    x_flat = x.reshape(rows_total)
    table_l = _tc_convert(table)
    gathered = _sc_gather(x_flat, table_l, rows_total, chunk=1280,
                          field1_off=_NUM_UNIQ[0])
    emb4 = gathered.reshape(n_rows // 4, 4 * F * _D)
    wt = W.T
    z = jnp.zeros_like(wt)
    w4 = jnp.block([
        [wt, z, z, z],
        [z, wt, z, z],
        [z, z, wt, z],
        [z, z, z, wt],
    ])
    b4 = jnp.tile(b, 4).reshape(1, 4 * _D)
    out4 = _tc_project(emb4, w4, b4, tile_n=1024)
    return out4.reshape(B, S, G, _D)


# speedup vs baseline: 2.2200x; 1.2119x over previous
import functools

import jax
import jax.numpy as jnp
from jax import lax
from jax.experimental import pallas as pl
from jax.experimental.pallas import tpu as pltpu
from jax.experimental.pallas import tpu_sc as plsc

_NUM_UNIQ = [1000000, 1000000]
_D = 32
_F = 2

_NC = 2
_NS = 16
_NW = _NC * _NS
_LANES = 16

_SUB = 128


def _sc_convert(table):
    v_rows = table.shape[0]
    slab_cols = 640
    n_slabs = v_rows // slab_cols
    per_w = -(-n_slabs // _NW)
    slab_out = slab_cols * _D
    tt = table.T

    mesh = plsc.VectorSubcoreMesh(core_axis_name="c", subcore_axis_name="s")

    @functools.partial(
        pl.kernel,
        mesh=mesh,
        out_type=jax.ShapeDtypeStruct((v_rows * _D,), jnp.float32),
        scratch_types=[
            pltpu.VMEM((_D, slab_cols), jnp.float32),
            pltpu.VMEM((slab_out,), jnp.float32),
            pltpu.VMEM((slab_out,), jnp.float32),
            pltpu.SemaphoreType.DMA,
        ],
        compiler_params=pltpu.CompilerParams(use_tc_tiling_on_sc=True,
                                             needs_layout_passes=False),
    )
    def conv_kernel(tt_hbm, out_hbm, in_v, out_a, out_b, sem):
        wid = lax.axis_index("s") * _NC + lax.axis_index("c")
        lanes = lax.iota(jnp.int32, 16)
        out_bufs = (out_a, out_b)

        def do_slab(slab, out_v):
            col0 = pl.multiple_of(slab * slab_cols, slab_cols)
            pltpu.sync_copy(tt_hbm.at[:, pl.ds(col0, slab_cols)], in_v)

            def grp_body(j, carry2):
                riv32 = (j * _LANES + lanes) * _D
                for c in range(_D):
                    vals = in_v[c, pl.ds(
                        pl.multiple_of(j * _LANES, _LANES), _LANES)]
                    plsc.store_scatter(out_v, [riv32 + c], vals)
                return carry2

            lax.fori_loop(0, slab_cols // _LANES, grp_body, 0)
            return pltpu.async_copy(
                out_v,
                out_hbm.at[pl.ds(pl.multiple_of(slab * slab_out, 1024),
                                 slab_out)],
                sem)

        def wait_out(i, buf):
            prev_slab = i * _NW + wid
            pltpu.make_async_copy(
                buf,
                out_hbm.at[pl.ds(
                    pl.multiple_of(prev_slab * slab_out, 1024), slab_out)],
                sem).wait()

        def it_body(k, carry):
            for u in range(2):
                i = k * 2 + u
                buf = out_bufs[u]
                slab = i * _NW + wid

                @pl.when(slab < n_slabs)
                def _(i=i, buf=buf, slab=slab):
                    @pl.when(i >= 2)
                    def _():
                        wait_out(i - 2, buf)
                    do_slab(slab, buf)
            return carry

        lax.fori_loop(0, per_w // 2, it_body, 0)
        for i in range(max(per_w - 3, 0), per_w):
            slab = i * _NW + wid
            nxt = (i + 2) * _NW + wid

            @pl.when((slab < n_slabs) & (nxt >= n_slabs))
            def _(i=i):
                wait_out(i, out_bufs[i % 2])

    return conv_kernel(tt).reshape(v_rows, _D)


def _tc_convert(table, block_cols=3200):
    v_rows = table.shape[0]
    tt = table.T
    n_blk = v_rows // block_cols
    rows_out = block_cols // 4
    eye = jnp.eye(_D, dtype=jnp.float32)

    def tr_kernel(tt_ref, eye_ref, out_ref):
        t = jax.lax.dot_general(
            tt_ref[...], eye_ref[...], (((0,), (0,)), ((), ())),
            preferred_element_type=jnp.float32)
        for k in range(block_cols // 128):
            for q in range(4):
                r0 = 128 * k + 32 * q
                out_ref[pl.ds(32 * k, 32), pl.ds(32 * q, 32)] = (
                    t[r0:r0 + 32, :]
                )

    out = pl.pallas_call(
        tr_kernel,
        grid=(n_blk,),
        in_specs=[
            pl.BlockSpec((_D, block_cols), lambda i: (0, i)),
            pl.BlockSpec((_D, _D), lambda i: (0, 0)),
        ],
        out_specs=pl.BlockSpec((rows_out, 4 * _D), lambda i: (i, 0)),
        out_shape=jax.ShapeDtypeStruct((v_rows // 4, 4 * _D), jnp.float32),
    )(tt, eye)
    return out.reshape(v_rows, _D)


def _sc_gather(x_flat, table_l, rows_total, chunk, field1_off):
    per_w = rows_total // _NW
    n_chunks = per_w // chunk
    n_sub = chunk // _SUB
    n_vec = chunk // _LANES

    mesh = plsc.VectorSubcoreMesh(core_axis_name="c", subcore_axis_name="s")

    @functools.partial(
        pl.kernel,
        mesh=mesh,
        out_type=jax.ShapeDtypeStruct((rows_total, _D), jnp.float32),
        scratch_types=[
            pltpu.VMEM((chunk,), jnp.int32),
            pltpu.VMEM((chunk, _D), jnp.float32),
            pltpu.SemaphoreType.DMA,
        ],
        compiler_params=pltpu.CompilerParams(use_tc_tiling_on_sc=False),
    )
    def gather_kernel(table_hbm, idx_hbm, out_hbm, idx_v, rows_v, sem):
        wid = lax.axis_index("s") * _NC + lax.axis_index("c")
        base = wid * per_w

        def chunk_body(i, carry):
            off = pl.multiple_of(base + i * chunk, _SUB)
            pltpu.sync_copy(idx_hbm.at[pl.ds(off, chunk)], idx_v)
            for j in range(n_vec):
                sl = pl.ds(j * _LANES, _LANES)
                fbit = (lax.shift_right_logical(off, 7) + (j // 8)) & 1
                r = idx_v[sl] + fbit * field1_off
                idx_v[sl] = (
                    (r & ~jnp.int32(127))
                    | lax.shift_left((r & 31), 2)
                    | (lax.shift_right_logical(r, 5) & 3)
                )
            handles = []
            for j in range(n_sub):
                handles.append(
                    pltpu.async_copy(
                        table_hbm.at[idx_v.at[pl.ds(j * _SUB, _SUB)]],
                        rows_v.at[pl.ds(j * _SUB, _SUB)],
                        sem,
                    )
                )
            for h in handles:
                h.wait()
            pltpu.sync_copy(rows_v, out_hbm.at[pl.ds(off, chunk)])
            return carry

        lax.fori_loop(0, n_chunks, chunk_body, 0)

    return gather_kernel(table_l, x_flat)


def _tc_project(emb4, w0b, w1b, b4, units):
    n4 = emb4.shape[0]

    def mm_kernel(emb_ref, w0_ref, w1_ref, b_ref, out_ref):
        for u in range(units):
            a0 = emb_ref[pl.ds(64 * u, 32), :]
            a1 = emb_ref[pl.ds(64 * u + 32, 32), :]
            o = (
                jnp.dot(a0, w0_ref[...], preferred_element_type=jnp.float32)
                + jnp.dot(a1, w1_ref[...],
                          preferred_element_type=jnp.float32)
                + b_ref[...]
            )
            out_ref[pl.ds(32 * u, 32), :] = o

    tile_in = 64 * units
    tile_out = 32 * units
    return pl.pallas_call(
        mm_kernel,
        grid=(n4 // tile_in,),
        in_specs=[
            pl.BlockSpec((tile_in, 128), lambda i: (i, 0)),
            pl.BlockSpec((128, 128), lambda i: (0, 0)),
            pl.BlockSpec((128, 128), lambda i: (0, 0)),
            pl.BlockSpec((1, 128), lambda i: (0, 0)),
        ],
        out_specs=pl.BlockSpec((tile_out, 128), lambda i: (i, 0)),
        out_shape=jax.ShapeDtypeStruct((n4 // 2, 128), jnp.float32),
    )(emb4, w0b, w1b, b4)


def kernel(x, table, W, b):
    B, S, G, F = x.shape
    n_rows = B * S * G
    rows_total = n_rows * F

    x_native = (x.reshape(32, 128, S, F).transpose((2, 0, 3, 1))
                .reshape(rows_total))
    table_l = _tc_convert(table)
    gathered = _sc_gather(x_native, table_l, rows_total, chunk=1280,
                          field1_off=_NUM_UNIQ[0])
    emb4 = gathered.reshape(rows_total // 4, 4 * _D)
    wt = W.T
    z = jnp.zeros((_D, _D), jnp.float32)
    w0t, w1t = wt[:_D], wt[_D:]
    w0b = jnp.block([
        [w0t, z, z, z],
        [z, w0t, z, z],
        [z, z, w0t, z],
        [z, z, z, w0t],
    ])
    w1b = jnp.block([
        [w1t, z, z, z],
        [z, w1t, z, z],
        [z, z, w1t, z],
        [z, z, z, w1t],
    ])
    b4 = jnp.tile(b, 4).reshape(1, 4 * _D)
    out4 = _tc_project(emb4, w0b, w1b, b4, units=16)
    return (out4.reshape(S, B, _D).transpose((1, 0, 2))
            .reshape(B, S, G, _D))


# speedup vs baseline: 2.7308x; 1.2301x over previous
import functools

import jax
import jax.numpy as jnp
from jax import lax
from jax.experimental import pallas as pl
from jax.experimental.pallas import tpu as pltpu
from jax.experimental.pallas import tpu_sc as plsc

_NUM_UNIQ = [1000000, 1000000]
_D = 32
_F = 2

_NC = 2
_NS = 16
_NW = _NC * _NS
_LANES = 16

_SUB = 128


def _sc_convert(table):
    v_rows = table.shape[0]
    slab_cols = 640
    n_slabs = v_rows // slab_cols
    per_w = -(-n_slabs // _NW)
    slab_out = slab_cols * _D
    tt = table.T

    mesh = plsc.VectorSubcoreMesh(core_axis_name="c", subcore_axis_name="s")

    @functools.partial(
        pl.kernel,
        mesh=mesh,
        out_type=jax.ShapeDtypeStruct((v_rows * _D,), jnp.float32),
        scratch_types=[
            pltpu.VMEM((_D, slab_cols), jnp.float32),
            pltpu.VMEM((slab_out,), jnp.float32),
            pltpu.VMEM((slab_out,), jnp.float32),
            pltpu.SemaphoreType.DMA,
        ],
        compiler_params=pltpu.CompilerParams(use_tc_tiling_on_sc=True,
                                             needs_layout_passes=False),
    )
    def conv_kernel(tt_hbm, out_hbm, in_v, out_a, out_b, sem):
        wid = lax.axis_index("s") * _NC + lax.axis_index("c")
        lanes = lax.iota(jnp.int32, 16)
        out_bufs = (out_a, out_b)

        def do_slab(slab, out_v):
            col0 = pl.multiple_of(slab * slab_cols, slab_cols)
            pltpu.sync_copy(tt_hbm.at[:, pl.ds(col0, slab_cols)], in_v)

            def grp_body(j, carry2):
                riv32 = (j * _LANES + lanes) * _D
                for c in range(_D):
                    vals = in_v[c, pl.ds(
                        pl.multiple_of(j * _LANES, _LANES), _LANES)]
                    plsc.store_scatter(out_v, [riv32 + c], vals)
                return carry2

            lax.fori_loop(0, slab_cols // _LANES, grp_body, 0)
            return pltpu.async_copy(
                out_v,
                out_hbm.at[pl.ds(pl.multiple_of(slab * slab_out, 1024),
                                 slab_out)],
                sem)

        def wait_out(i, buf):
            prev_slab = i * _NW + wid
            pltpu.make_async_copy(
                buf,
                out_hbm.at[pl.ds(
                    pl.multiple_of(prev_slab * slab_out, 1024), slab_out)],
                sem).wait()

        def it_body(k, carry):
            for u in range(2):
                i = k * 2 + u
                buf = out_bufs[u]
                slab = i * _NW + wid

                @pl.when(slab < n_slabs)
                def _(i=i, buf=buf, slab=slab):
                    @pl.when(i >= 2)
                    def _():
                        wait_out(i - 2, buf)
                    do_slab(slab, buf)
            return carry

        lax.fori_loop(0, per_w // 2, it_body, 0)
        for i in range(max(per_w - 3, 0), per_w):
            slab = i * _NW + wid
            nxt = (i + 2) * _NW + wid

            @pl.when((slab < n_slabs) & (nxt >= n_slabs))
            def _(i=i):
                wait_out(i, out_bufs[i % 2])

    return conv_kernel(tt).reshape(v_rows, _D)


def _tc_convert(table, block_cols=16000):
    v_rows = table.shape[0]
    tt = table.T
    n_blk = v_rows // block_cols
    rows_out = block_cols // 4
    eye = jnp.eye(_D, dtype=jnp.float32)

    def tr_kernel(tt_ref, eye_ref, out_ref):
        t = jax.lax.dot_general(
            tt_ref[...], eye_ref[...], (((0,), (0,)), ((), ())),
            preferred_element_type=jnp.float32)
        for k in range(block_cols // 128):
            for q in range(4):
                r0 = 128 * k + 32 * q
                out_ref[pl.ds(32 * k, 32), pl.ds(32 * q, 32)] = (
                    t[r0:r0 + 32, :]
                )

    out = pl.pallas_call(
        tr_kernel,
        grid=(n_blk,),
        in_specs=[
            pl.BlockSpec((_D, block_cols), lambda i: (0, i)),
            pl.BlockSpec((_D, _D), lambda i: (0, 0)),
        ],
        out_specs=pl.BlockSpec((rows_out, 4 * _D), lambda i: (i, 0)),
        out_shape=jax.ShapeDtypeStruct((v_rows // 4, 4 * _D), jnp.float32),
    )(tt, eye)
    return out.reshape(v_rows, _D)


def _sc_gather(x_flat, table_l, rows_total, chunk, field1_off):
    per_w = rows_total // _NW
    n_chunks = per_w // chunk
    n_sub = chunk // _SUB
    n_vec = chunk // _LANES

    mesh = plsc.VectorSubcoreMesh(core_axis_name="c", subcore_axis_name="s")

    @functools.partial(
        pl.kernel,
        mesh=mesh,
        out_type=jax.ShapeDtypeStruct((rows_total, _D), jnp.float32),
        scratch_types=[
            pltpu.VMEM((chunk,), jnp.int32),
            pltpu.VMEM((chunk, _D), jnp.float32),
            pltpu.SemaphoreType.DMA,
        ],
        compiler_params=pltpu.CompilerParams(use_tc_tiling_on_sc=False),
    )
    def gather_kernel(table_hbm, idx_hbm, out_hbm, idx_v, rows_v, sem):
        wid = lax.axis_index("s") * _NC + lax.axis_index("c")
        base = wid * per_w

        def chunk_body(i, carry):
            off = pl.multiple_of(base + i * chunk, _SUB)
            pltpu.sync_copy(idx_hbm.at[pl.ds(off, chunk)], idx_v)
            for j in range(n_vec):
                sl = pl.ds(j * _LANES, _LANES)
                fbit = (lax.shift_right_logical(off, 7) + (j // 8)) & 1
                r = idx_v[sl] + fbit * field1_off
                idx_v[sl] = (
                    (r & ~jnp.int32(127))
                    | lax.shift_left((r & 31), 2)
                    | (lax.shift_right_logical(r, 5) & 3)
                )
            handles = []
            for j in range(n_sub):
                handles.append(
                    pltpu.async_copy(
                        table_hbm.at[idx_v.at[pl.ds(j * _SUB, _SUB)]],
                        rows_v.at[pl.ds(j * _SUB, _SUB)],
                        sem,
                    )
                )
            for h in handles:
                h.wait()
            pltpu.sync_copy(rows_v, out_hbm.at[pl.ds(off, chunk)])
            return carry

        lax.fori_loop(0, n_chunks, chunk_body, 0)

    return gather_kernel(table_l, x_flat)


def _tc_project(emb4, w0b, w1b, b4, units):
    n4 = emb4.shape[0]

    def mm_kernel(emb_ref, w0_ref, w1_ref, b_ref, out_ref):
        for u in range(units):
            a0 = emb_ref[pl.ds(64 * u, 32), :]
            a1 = emb_ref[pl.ds(64 * u + 32, 32), :]
            o = (
                jnp.dot(a0, w0_ref[...], preferred_element_type=jnp.float32)
                + jnp.dot(a1, w1_ref[...],
                          preferred_element_type=jnp.float32)
                + b_ref[...]
            )
            out_ref[pl.ds(32 * u, 32), :] = o

    tile_in = 64 * units
    tile_out = 32 * units
    return pl.pallas_call(
        mm_kernel,
        grid=(n4 // tile_in,),
        in_specs=[
            pl.BlockSpec((tile_in, 128), lambda i: (i, 0)),
            pl.BlockSpec((128, 128), lambda i: (0, 0)),
            pl.BlockSpec((128, 128), lambda i: (0, 0)),
            pl.BlockSpec((1, 128), lambda i: (0, 0)),
        ],
        out_specs=pl.BlockSpec((tile_out, 128), lambda i: (i, 0)),
        out_shape=jax.ShapeDtypeStruct((n4 // 2, 128), jnp.float32),
    )(emb4, w0b, w1b, b4)


def kernel(x, table, W, b):
    B, S, G, F = x.shape
    n_rows = B * S * G
    rows_total = n_rows * F

    x_native = (x.reshape(32, 128, S, F).transpose((2, 0, 3, 1))
                .reshape(rows_total))
    table_l = _tc_convert(table)
    gathered = _sc_gather(x_native, table_l, rows_total, chunk=1280,
                          field1_off=_NUM_UNIQ[0])
    emb4 = gathered.reshape(rows_total // 4, 4 * _D)
    wt = W.T
    z = jnp.zeros((_D, _D), jnp.float32)
    w0t, w1t = wt[:_D], wt[_D:]
    w0b = jnp.block([
        [w0t, z, z, z],
        [z, w0t, z, z],
        [z, z, w0t, z],
        [z, z, z, w0t],
    ])
    w1b = jnp.block([
        [w1t, z, z, z],
        [z, w1t, z, z],
        [z, z, w1t, z],
        [z, z, z, w1t],
    ])
    b4 = jnp.tile(b, 4).reshape(1, 4 * _D)
    out4 = _tc_project(emb4, w0b, w1b, b4, units=16)
    return (out4.reshape(S, B, _D).transpose((1, 0, 2))
            .reshape(B, S, G, _D))


# speedup vs baseline: 2.7436x; 1.0047x over previous
import functools

import jax
import jax.numpy as jnp
from jax import lax
from jax.experimental import pallas as pl
from jax.experimental.pallas import tpu as pltpu
from jax.experimental.pallas import tpu_sc as plsc

_NUM_UNIQ = [1000000, 1000000]
_D = 32
_F = 2

_NC = 2
_NS = 16
_NW = _NC * _NS
_LANES = 16

_SUB = 128


def _sc_convert(table):
    v_rows = table.shape[0]
    slab_cols = 640
    n_slabs = v_rows // slab_cols
    per_w = -(-n_slabs // _NW)
    slab_out = slab_cols * _D
    tt = table.T

    mesh = plsc.VectorSubcoreMesh(core_axis_name="c", subcore_axis_name="s")

    @functools.partial(
        pl.kernel,
        mesh=mesh,
        out_type=jax.ShapeDtypeStruct((v_rows * _D,), jnp.float32),
        scratch_types=[
            pltpu.VMEM((_D, slab_cols), jnp.float32),
            pltpu.VMEM((slab_out,), jnp.float32),
            pltpu.VMEM((slab_out,), jnp.float32),
            pltpu.SemaphoreType.DMA,
        ],
        compiler_params=pltpu.CompilerParams(use_tc_tiling_on_sc=True,
                                             needs_layout_passes=False),
    )
    def conv_kernel(tt_hbm, out_hbm, in_v, out_a, out_b, sem):
        wid = lax.axis_index("s") * _NC + lax.axis_index("c")
        lanes = lax.iota(jnp.int32, 16)
        out_bufs = (out_a, out_b)

        def do_slab(slab, out_v):
            col0 = pl.multiple_of(slab * slab_cols, slab_cols)
            pltpu.sync_copy(tt_hbm.at[:, pl.ds(col0, slab_cols)], in_v)

            def grp_body(j, carry2):
                riv32 = (j * _LANES + lanes) * _D
                for c in range(_D):
                    vals = in_v[c, pl.ds(
                        pl.multiple_of(j * _LANES, _LANES), _LANES)]
                    plsc.store_scatter(out_v, [riv32 + c], vals)
                return carry2

            lax.fori_loop(0, slab_cols // _LANES, grp_body, 0)
            return pltpu.async_copy(
                out_v,
                out_hbm.at[pl.ds(pl.multiple_of(slab * slab_out, 1024),
                                 slab_out)],
                sem)

        def wait_out(i, buf):
            prev_slab = i * _NW + wid
            pltpu.make_async_copy(
                buf,
                out_hbm.at[pl.ds(
                    pl.multiple_of(prev_slab * slab_out, 1024), slab_out)],
                sem).wait()

        def it_body(k, carry):
            for u in range(2):
                i = k * 2 + u
                buf = out_bufs[u]
                slab = i * _NW + wid

                @pl.when(slab < n_slabs)
                def _(i=i, buf=buf, slab=slab):
                    @pl.when(i >= 2)
                    def _():
                        wait_out(i - 2, buf)
                    do_slab(slab, buf)
            return carry

        lax.fori_loop(0, per_w // 2, it_body, 0)
        for i in range(max(per_w - 3, 0), per_w):
            slab = i * _NW + wid
            nxt = (i + 2) * _NW + wid

            @pl.when((slab < n_slabs) & (nxt >= n_slabs))
            def _(i=i):
                wait_out(i, out_bufs[i % 2])

    return conv_kernel(tt).reshape(v_rows, _D)


def _tc_convert(table, block_cols=16000):
    v_rows = table.shape[0]
    tt = table.T
    n_blk = v_rows // block_cols
    rows_out = block_cols // 4
    eye = jnp.eye(_D, dtype=jnp.float32)

    def tr_kernel(tt_ref, eye_ref, out_ref):
        t = jax.lax.dot_general(
            tt_ref[...], eye_ref[...], (((0,), (0,)), ((), ())),
            preferred_element_type=jnp.float32)
        for k in range(block_cols // 128):
            r0 = 128 * k
            tile = jnp.concatenate(
                [t[r0 + 32 * q:r0 + 32 * (q + 1), :] for q in range(4)],
                axis=1)
            out_ref[pl.ds(32 * k, 32), :] = tile

    out = pl.pallas_call(
        tr_kernel,
        grid=(n_blk,),
        in_specs=[
            pl.BlockSpec((_D, block_cols), lambda i: (0, i)),
            pl.BlockSpec((_D, _D), lambda i: (0, 0)),
        ],
        out_specs=pl.BlockSpec((rows_out, 4 * _D), lambda i: (i, 0)),
        out_shape=jax.ShapeDtypeStruct((v_rows // 4, 4 * _D), jnp.float32),
    )(tt, eye)
    return out.reshape(v_rows, _D)


def _sc_gather(x_flat, table_l, rows_total, chunk, field1_off):
    per_w = rows_total // _NW
    n_chunks = per_w // chunk
    n_sub = chunk // _SUB
    n_vec = chunk // _LANES

    mesh = plsc.VectorSubcoreMesh(core_axis_name="c", subcore_axis_name="s")

    @functools.partial(
        pl.kernel,
        mesh=mesh,
        out_type=jax.ShapeDtypeStruct((rows_total, _D), jnp.float32),
        scratch_types=[
            pltpu.VMEM((chunk,), jnp.int32),
            pltpu.VMEM((chunk, _D), jnp.float32),
            pltpu.SemaphoreType.DMA,
        ],
        compiler_params=pltpu.CompilerParams(use_tc_tiling_on_sc=False),
    )
    def gather_kernel(table_hbm, idx_hbm, out_hbm, idx_v, rows_v, sem):
        wid = lax.axis_index("s") * _NC + lax.axis_index("c")
        base = wid * per_w

        def chunk_body(i, carry):
            off = pl.multiple_of(base + i * chunk, _SUB)
            pltpu.sync_copy(idx_hbm.at[pl.ds(off, chunk)], idx_v)
            for j in range(n_vec):
                sl = pl.ds(j * _LANES, _LANES)
                fbit = (lax.shift_right_logical(off, 7) + (j // 8)) & 1
                r = idx_v[sl] + fbit * field1_off
                idx_v[sl] = (
                    (r & ~jnp.int32(127))
                    | lax.shift_left((r & 31), 2)
                    | (lax.shift_right_logical(r, 5) & 3)
                )
            handles = []
            for j in range(n_sub):
                handles.append(
                    pltpu.async_copy(
                        table_hbm.at[idx_v.at[pl.ds(j * _SUB, _SUB)]],
                        rows_v.at[pl.ds(j * _SUB, _SUB)],
                        sem,
                    )
                )
            for h in handles:
                h.wait()
            pltpu.sync_copy(rows_v, out_hbm.at[pl.ds(off, chunk)])
            return carry

        lax.fori_loop(0, n_chunks, chunk_body, 0)

    return gather_kernel(table_l, x_flat)


def _tc_project(emb4, w0b, w1b, b4, units):
    n4 = emb4.shape[0]

    def mm_kernel(emb_ref, w0_ref, w1_ref, b_ref, out_ref):
        a = emb_ref[...]
        o0 = jnp.dot(a, w0_ref[...], preferred_element_type=jnp.float32)
        o1 = jnp.dot(a, w1_ref[...], preferred_element_type=jnp.float32)
        for u in range(units):
            o = (o0[64 * u:64 * u + 32, :]
                 + o1[64 * u + 32:64 * u + 64, :] + b_ref[...])
            out_ref[pl.ds(32 * u, 32), :] = o

    tile_in = 64 * units
    tile_out = 32 * units
    return pl.pallas_call(
        mm_kernel,
        grid=(n4 // tile_in,),
        in_specs=[
            pl.BlockSpec((tile_in, 128), lambda i: (i, 0)),
            pl.BlockSpec((128, 128), lambda i: (0, 0)),
            pl.BlockSpec((128, 128), lambda i: (0, 0)),
            pl.BlockSpec((1, 128), lambda i: (0, 0)),
        ],
        out_specs=pl.BlockSpec((tile_out, 128), lambda i: (i, 0)),
        out_shape=jax.ShapeDtypeStruct((n4 // 2, 128), jnp.float32),
    )(emb4, w0b, w1b, b4)


def kernel(x, table, W, b):
    B, S, G, F = x.shape
    n_rows = B * S * G
    rows_total = n_rows * F

    x_native = (x.reshape(32, 128, S, F).transpose((2, 0, 3, 1))
                .reshape(rows_total))
    table_l = _tc_convert(table)
    gathered = _sc_gather(x_native, table_l, rows_total, chunk=1280,
                          field1_off=_NUM_UNIQ[0])
    emb4 = gathered.reshape(rows_total // 4, 4 * _D)
    wt = W.T
    z = jnp.zeros((_D, _D), jnp.float32)
    w0t, w1t = wt[:_D], wt[_D:]
    w0b = jnp.block([
        [w0t, z, z, z],
        [z, w0t, z, z],
        [z, z, w0t, z],
        [z, z, z, w0t],
    ])
    w1b = jnp.block([
        [w1t, z, z, z],
        [z, w1t, z, z],
        [z, z, w1t, z],
        [z, z, z, w1t],
    ])
    b4 = jnp.tile(b, 4).reshape(1, 4 * _D)
    out4 = _tc_project(emb4, w0b, w1b, b4, units=16)
    return (out4.reshape(S, B, _D).transpose((1, 0, 2))
            .reshape(B, S, G, _D))


# speedup vs baseline: 2.7438x; 1.0001x over previous
import functools

import jax
import jax.numpy as jnp
from jax import lax
from jax.experimental import pallas as pl
from jax.experimental.pallas import tpu as pltpu
from jax.experimental.pallas import tpu_sc as plsc

_NUM_UNIQ = [1000000, 1000000]
_D = 32
_F = 2

_NC = 2
_NS = 16
_NW = _NC * _NS
_LANES = 16

_SUB = 128


def _tc_convert(table, block_cols=16000):
    v_rows = table.shape[0]
    tt = table.T
    n_blk = v_rows // block_cols
    rows_out = block_cols // 4
    eye = jnp.eye(_D, dtype=jnp.float32)

    def tr_kernel(tt_ref, eye_ref, out_ref):
        t = jax.lax.dot_general(
            tt_ref[...], eye_ref[...], (((0,), (0,)), ((), ())),
            preferred_element_type=jnp.float32)
        for k in range(block_cols // 128):
            r0 = 128 * k
            tile = jnp.concatenate(
                [t[r0 + 32 * q:r0 + 32 * (q + 1), :] for q in range(4)],
                axis=1)
            out_ref[pl.ds(32 * k, 32), :] = tile

    out = pl.pallas_call(
        tr_kernel,
        grid=(n_blk,),
        in_specs=[
            pl.BlockSpec((_D, block_cols), lambda i: (0, i)),
            pl.BlockSpec((_D, _D), lambda i: (0, 0)),
        ],
        out_specs=pl.BlockSpec((rows_out, 4 * _D), lambda i: (i, 0)),
        out_shape=jax.ShapeDtypeStruct((v_rows // 4, 4 * _D), jnp.float32),
    )(tt, eye)
    return out.reshape(v_rows, _D)


def _sc_gather(x_flat, table_l, rows_total, chunk, field1_off):
    per_w = rows_total // _NW
    n_chunks = per_w // chunk
    n_sub = chunk // _SUB
    n_vec = chunk // _LANES

    mesh = plsc.VectorSubcoreMesh(core_axis_name="c", subcore_axis_name="s")

    @functools.partial(
        pl.kernel,
        mesh=mesh,
        out_type=jax.ShapeDtypeStruct((rows_total, _D), jnp.float32),
        scratch_types=[
            pltpu.VMEM((chunk,), jnp.int32),
            pltpu.VMEM((chunk, _D), jnp.float32),
            pltpu.SemaphoreType.DMA,
        ],
        compiler_params=pltpu.CompilerParams(use_tc_tiling_on_sc=False),
    )
    def gather_kernel(table_hbm, idx_hbm, out_hbm, idx_v, rows_v, sem):
        wid = lax.axis_index("s") * _NC + lax.axis_index("c")
        base = wid * per_w

        def chunk_body(i, carry):
            off = pl.multiple_of(base + i * chunk, _SUB)
            pltpu.sync_copy(idx_hbm.at[pl.ds(off, chunk)], idx_v)
            for j in range(n_vec):
                sl = pl.ds(j * _LANES, _LANES)
                fbit = (lax.shift_right_logical(off, 7) + (j // 8)) & 1
                r = idx_v[sl] + fbit * field1_off
                idx_v[sl] = (
                    (r & ~jnp.int32(127))
                    | lax.shift_left((r & 31), 2)
                    | (lax.shift_right_logical(r, 5) & 3)
                )
            handles = []
            for j in range(n_sub):
                handles.append(
                    pltpu.async_copy(
                        table_hbm.at[idx_v.at[pl.ds(j * _SUB, _SUB)]],
                        rows_v.at[pl.ds(j * _SUB, _SUB)],
                        sem,
                    )
                )
            for h in handles:
                h.wait()
            pltpu.sync_copy(rows_v, out_hbm.at[pl.ds(off, chunk)])
            return carry

        lax.fori_loop(0, n_chunks, chunk_body, 0)

    return gather_kernel(table_l, x_flat)


def _tc_project(emb4, w0b, w1b, b4, units):
    n4 = emb4.shape[0]

    def mm_kernel(emb_ref, w0_ref, w1_ref, b_ref, out_ref):
        a = emb_ref[...]
        o0 = jnp.dot(a, w0_ref[...], preferred_element_type=jnp.float32)
        o1 = jnp.dot(a, w1_ref[...], preferred_element_type=jnp.float32)
        for u in range(units):
            o = (o0[64 * u:64 * u + 32, :]
                 + o1[64 * u + 32:64 * u + 64, :] + b_ref[...])
            out_ref[pl.ds(32 * u, 32), :] = o

    tile_in = 64 * units
    tile_out = 32 * units
    return pl.pallas_call(
        mm_kernel,
        grid=(n4 // tile_in,),
        in_specs=[
            pl.BlockSpec((tile_in, 128), lambda i: (i, 0)),
            pl.BlockSpec((128, 128), lambda i: (0, 0)),
            pl.BlockSpec((128, 128), lambda i: (0, 0)),
            pl.BlockSpec((1, 128), lambda i: (0, 0)),
        ],
        out_specs=pl.BlockSpec((tile_out, 128), lambda i: (i, 0)),
        out_shape=jax.ShapeDtypeStruct((n4 // 2, 128), jnp.float32),
    )(emb4, w0b, w1b, b4)


def kernel(x, table, W, b):
    B, S, G, F = x.shape
    n_rows = B * S * G
    rows_total = n_rows * F

    x_native = (x.reshape(32, 128, S, F).transpose((2, 0, 3, 1))
                .reshape(rows_total))
    table_l = _tc_convert(table)
    gathered = _sc_gather(x_native, table_l, rows_total, chunk=1280,
                          field1_off=_NUM_UNIQ[0])
    emb4 = gathered.reshape(rows_total // 4, 4 * _D)
    wt = W.T
    z = jnp.zeros((_D, _D), jnp.float32)
    w0t, w1t = wt[:_D], wt[_D:]
    w0b = jnp.block([
        [w0t, z, z, z],
        [z, w0t, z, z],
        [z, z, w0t, z],
        [z, z, z, w0t],
    ])
    w1b = jnp.block([
        [w1t, z, z, z],
        [z, w1t, z, z],
        [z, z, w1t, z],
        [z, z, z, w1t],
    ])
    b4 = jnp.tile(b, 4).reshape(1, 4 * _D)
    out4 = _tc_project(emb4, w0b, w1b, b4, units=16)
    return (out4.reshape(S, B, _D).transpose((1, 0, 2))
            .reshape(B, S, G, _D))
